# baseline (speedup 1.0000x reference)
import jax
import jax.numpy as jnp
from jax import lax
from jax.experimental import pallas as pl
from jax.experimental.pallas import tpu as pltpu

_Q = 1024
_NC = 1000
_NCP = 1024
_D = 64
_K = 16
_BQ = 256


def _dist_knn_kernel(x_ref, c_ref, dist_hbm, knn_ref, scratch, sem):
    i = pl.program_id(0)
    xb = x_ref[...]
    cb = c_ref[...]
    xn = jnp.sum(xb * xb, axis=1, keepdims=True)
    cn = jnp.sum(cb * cb, axis=1, keepdims=True)
    cross = lax.dot_general(xb, cb, (((1,), (1,)), ((), ())),
                            precision=lax.Precision.HIGHEST)
    dist = (xn - 2.0 * cross) + cn.T
    inf = jnp.float32(jnp.inf)
    work = jnp.concatenate(
        [dist, jnp.full((_BQ, _NCP - _NC), inf, jnp.float32)], axis=1)
    scratch[...] = work
    cp = pltpu.make_async_copy(
        scratch, dist_hbm.at[pl.ds(i * _BQ, _BQ), :], sem)
    cp.start()

    fiota = lax.broadcasted_iota(jnp.int32, (_BQ, _NCP), 1).astype(jnp.float32)
    mval = jnp.min(work, axis=1, keepdims=True)
    cols = []
    for j in range(_K):
        midx = jnp.min(jnp.where(work == mval, fiota, inf),
                       axis=1, keepdims=True)
        cols.append(midx)
        if j < _K - 1:
            work = jnp.where(fiota == midx, inf, work)
            mval = jnp.min(work, axis=1, keepdims=True)
    knn_ref[...] = jnp.concatenate(cols, axis=1).astype(jnp.int32)
    cp.wait()


def kernel(x, centers, k):
    del k
    dist_padded, knn = pl.pallas_call(
        _dist_knn_kernel,
        grid=(_Q // _BQ,),
        in_specs=[
            pl.BlockSpec((_BQ, _D), lambda i: (i, 0)),
            pl.BlockSpec((_NC, _D), lambda i: (0, 0)),
        ],
        out_specs=[
            pl.BlockSpec(memory_space=pl.ANY),
            pl.BlockSpec((_BQ, _K), lambda i: (i, 0)),
        ],
        out_shape=[
            jax.ShapeDtypeStruct((_Q, _NCP), jnp.float32),
            jax.ShapeDtypeStruct((_Q, _K), jnp.int32),
        ],
        scratch_shapes=[
            pltpu.VMEM((_BQ, _NCP), jnp.float32),
            pltpu.SemaphoreType.DMA,
        ],
    )(x, centers)
    return dist_padded[:, :_NC], knn


# speedup vs baseline: 1.0061x; 1.0061x over previous
import jax
import jax.numpy as jnp
from jax import lax
from jax.experimental import pallas as pl

_Q = 1024
_NC = 1000
_NCP = 1024
_D = 64
_K = 16
_BQ = 512


def _dist_knn_kernel(x_ref, c_ref, dist_ref, knn_ref):
    xb = x_ref[...]
    cb = c_ref[...]
    xn = jnp.sum(xb * xb, axis=1, keepdims=True)
    cn = jnp.sum(cb * cb, axis=1, keepdims=True)
    cross = lax.dot_general(xb, cb, (((1,), (1,)), ((), ())),
                            precision=lax.Precision.HIGHEST)
    dist = (xn - 2.0 * cross) + cn.T
    inf = jnp.float32(jnp.inf)
    work = jnp.concatenate(
        [dist, jnp.full((_BQ, _NCP - _NC), inf, jnp.float32)], axis=1)
    dist_ref[...] = work

    fiota = lax.broadcasted_iota(jnp.int32, (_BQ, _NCP), 1).astype(jnp.float32)
    mval = jnp.min(work, axis=1, keepdims=True)
    cols = []
    for j in range(_K):
        midx = jnp.min(jnp.where(work == mval, fiota, inf),
                       axis=1, keepdims=True)
        cols.append(midx)
        if j < _K - 1:
            work = jnp.where(fiota == midx, inf, work)
            mval = jnp.min(work, axis=1, keepdims=True)
    knn_ref[...] = jnp.concatenate(cols, axis=1).astype(jnp.int32)


def kernel(x, centers, k):
    del k
    dist_padded, knn = pl.pallas_call(
        _dist_knn_kernel,
        grid=(_Q // _BQ,),
        in_specs=[
            pl.BlockSpec((_BQ, _D), lambda i: (i, 0)),
            pl.BlockSpec((_NC, _D), lambda i: (0, 0)),
        ],
        out_specs=[
            pl.BlockSpec((_BQ, _NCP), lambda i: (i, 0)),
            pl.BlockSpec((_BQ, _K), lambda i: (i, 0)),
        ],
        out_shape=[
            jax.ShapeDtypeStruct((_Q, _NCP), jnp.float32),
            jax.ShapeDtypeStruct((_Q, _K), jnp.int32),
        ],
    )(x, centers)
    return dist_padded[:, :_NC], knn
